# baseline (speedup 1.0000x reference)
import jax
import jax.numpy as jnp
from jax import lax
from jax.experimental import pallas as pl
from jax.experimental.pallas import tpu as pltpu
from jax.experimental.pallas import tpu_sc as plsc

N = 2048
C = 64
F = 256
NSUB = 16
NCORE = 2
CPT = 4
RPT = 2
PG = 16
NR = 3200
MYCAP = N + RPT * (PG - 1) + PG
TM = 128


def _route_body(ids_hbm, x_hbm, perm_hbm, off_hbm, nblk_hbm, pcnt_hbm, xr_hbm,
                ids_v, mypx_v, mypy_v, cntg_v, off_v, nblk_v, pcnt_v,
                tmp16_v, idx128_v, rows128_v, idx16_v, rows16_v,
                cnt_sh, sem):
    cid = lax.axis_index("c")
    sid = lax.axis_index("s")
    lane = lax.iota(jnp.int32, 16)
    zeros16 = jnp.zeros((16,), jnp.int32)
    c0 = CPT * sid
    r0 = CPT * sid + RPT * cid

    pltpu.sync_copy(ids_hbm, ids_v)

    def cnt_body(k, acc):
        ids = ids_v[pl.ds(k * 16, 16)]
        return tuple(
            acc[j] + plsc.all_reduce_population_count(ids == (c0 + j))
            for j in range(CPT))

    accs = lax.fori_loop(0, N // 16, cnt_body,
                         tuple(zeros16 for _ in range(CPT)))
    row = zeros16
    for j in range(CPT):
        row = jnp.where(lane == j, accs[j], row)
    tmp16_v[...] = row
    pltpu.sync_copy(tmp16_v, cnt_sh.at[pl.ds(sid * 16, 16)])
    plsc.subcore_barrier()

    pltpu.sync_copy(cnt_sh, cntg_v)
    carry = jnp.int32(0)
    for g in range(C // 16):
        flat_idx = ((4 * g + jnp.right_shift(lane, 2)) * 16
                    + jnp.bitwise_and(lane, 3))
        cnt = plsc.load_gather(cntg_v, [flat_idx])
        pcnt = jnp.bitwise_and(cnt + (PG - 1), jnp.int32(-PG))
        cum = plsc.cumsum(pcnt)
        off = cum - pcnt + carry
        nblk = jnp.right_shift(cnt + (TM - 1), 7)
        off_v[pl.ds(16 * g, 16)] = off
        nblk_v[pl.ds(16 * g, 16)] = nblk
        pcnt_v[pl.ds(16 * g, 16)] = pcnt
        carry = carry + jnp.sum(pcnt)

    @pl.when(jnp.logical_and(sid == 0, cid == 0))
    def _write_meta():
        pltpu.sync_copy(off_v, off_hbm)
        pltpu.sync_copy(nblk_v, nblk_hbm)
        pltpu.sync_copy(pcnt_v, pcnt_hbm)

    bases0 = tuple(plsc.load_gather(off_v, [zeros16 + (r0 + j)])
                   for j in range(RPT))
    my_pc = [plsc.load_gather(pcnt_v, [zeros16 + (r0 + j)])
             for j in range(RPT)]
    b0 = pl.multiple_of(jnp.sum(jnp.where(lane == 0, bases0[0], 0)), PG)

    def pos_body(k, bases):
        ids = ids_v[pl.ds(k * 16, 16)]
        tok = k * 16 + lane
        new_bases = []
        for j in range(RPT):
            m = ids == (r0 + j)
            incl = plsc.cumsum(jnp.where(m, 1, 0))
            rel = (bases[j] - b0) + incl - 1
            plsc.store_scatter(mypx_v, [rel], tok, mask=m)
            plsc.store_scatter(mypy_v, [rel], tok, mask=m)
            new_bases.append(bases[j] + plsc.all_reduce_population_count(m))
        return tuple(new_bases)

    ends = lax.fori_loop(0, N // 16, pos_body, bases0)

    for j in range(RPT):
        rel = (ends[j] - b0) + lane
        npad = (bases0[j] + my_pc[j]) - ends[j]
        plsc.store_scatter(mypx_v, [rel], zeros16, mask=lane < npad)
        plsc.store_scatter(mypy_v, [rel], zeros16 + N, mask=lane < npad)

    mylen = jnp.sum(jnp.where(lane == 0, my_pc[0] + my_pc[1], 0))
    nfull = jnp.right_shift(mylen, 7)
    ntail = jnp.right_shift(jnp.bitwise_and(mylen, TM - 1), 4)

    def full_body(i, _):
        s = pl.multiple_of(i * TM, PG)
        for q in range(TM // 16):
            idx128_v[pl.ds(q * 16, 16)] = mypx_v[pl.ds(s + q * 16, 16)]
        t = pl.multiple_of(b0 + s, PG)
        pltpu.async_copy(x_hbm.at[idx128_v], rows128_v, sem).wait()
        pltpu.sync_copy(rows128_v, xr_hbm.at[pl.ds(t, TM)])
        for q in range(TM // 16):
            idx128_v[pl.ds(q * 16, 16)] = mypy_v[pl.ds(s + q * 16, 16)]
        pltpu.sync_copy(idx128_v, perm_hbm.at[pl.ds(t, TM)])
        return 0

    lax.fori_loop(0, nfull, full_body, 0)

    def tail_body(i, _):
        s = pl.multiple_of(nfull * TM + i * PG, PG)
        t = pl.multiple_of(b0 + s, PG)
        idx16_v[...] = mypx_v[pl.ds(s, 16)]
        pltpu.async_copy(x_hbm.at[idx16_v], rows16_v, sem).wait()
        pltpu.sync_copy(rows16_v, xr_hbm.at[pl.ds(t, PG)])
        idx16_v[...] = mypy_v[pl.ds(s, 16)]
        pltpu.sync_copy(idx16_v, perm_hbm.at[pl.ds(t, PG)])
        return 0

    lax.fori_loop(0, ntail, tail_body, 0)


def _unroute_body(perm_hbm, off_hbm, pcnt_hbm, yr_hbm, y_hbm,
                  off_v, pcnt_v, idx128_v, rows128_v, idx16_v, rows16_v, sem):
    cid = lax.axis_index("c")
    sid = lax.axis_index("s")
    lane = lax.iota(jnp.int32, 16)
    r0 = CPT * sid + RPT * cid
    pltpu.sync_copy(off_hbm, off_v)
    pltpu.sync_copy(pcnt_hbm, pcnt_v)
    myoff = plsc.load_gather(off_v, [r0 + jnp.bitwise_and(lane, 1)])
    mypc = plsc.load_gather(pcnt_v, [r0 + jnp.bitwise_and(lane, 1)])
    b0 = pl.multiple_of(jnp.sum(jnp.where(lane == 0, myoff, 0)), PG)
    mylen = jnp.sum(jnp.where(lane < RPT, mypc, 0))
    nfull = jnp.right_shift(mylen, 7)
    ntail = jnp.right_shift(jnp.bitwise_and(mylen, TM - 1), 4)

    def full_body(i, _):
        s = pl.multiple_of(b0 + i * TM, PG)
        pltpu.sync_copy(perm_hbm.at[pl.ds(s, TM)], idx128_v)
        pltpu.sync_copy(yr_hbm.at[pl.ds(s, TM)], rows128_v)
        pltpu.async_copy(rows128_v, y_hbm.at[idx128_v], sem).wait()
        return 0

    lax.fori_loop(0, nfull, full_body, 0)

    def tail_body(i, _):
        s = pl.multiple_of(b0 + nfull * TM + i * PG, PG)
        pltpu.sync_copy(perm_hbm.at[pl.ds(s, PG)], idx16_v)
        pltpu.sync_copy(yr_hbm.at[pl.ds(s, PG)], rows16_v)
        pltpu.async_copy(rows16_v, y_hbm.at[idx16_v], sem).wait()
        return 0

    lax.fori_loop(0, ntail, tail_body, 0)


def _mm_body(off_ref, nblk_ref, xr_ref, w_ref, b_ref, o_ref):
    c = pl.program_id(0)
    start = pl.multiple_of(off_ref[c], 8)
    n = nblk_ref[c]
    wcat = w_ref[0].astype(jnp.bfloat16)
    brow = b_ref[0]

    def body(i, _):
        rows = xr_ref[pl.ds(start + i * TM, TM), :]
        acc = jnp.dot(rows.astype(jnp.bfloat16), wcat,
                      preferred_element_type=jnp.float32)
        o_ref[pl.ds(start + i * TM, TM), :] = acc + brow
        return 0

    lax.fori_loop(0, n, body, 0)


def _sc_mesh():
    return plsc.VectorSubcoreMesh(core_axis_name="c", subcore_axis_name="s",
                                  num_cores=NCORE)


def kernel(x, cat_ids, W, b):
    ids = cat_ids.astype(jnp.int32)

    route = pl.kernel(
        _route_body,
        out_type=[
            jax.ShapeDtypeStruct((NR,), jnp.int32),
            jax.ShapeDtypeStruct((C,), jnp.int32),
            jax.ShapeDtypeStruct((C,), jnp.int32),
            jax.ShapeDtypeStruct((C,), jnp.int32),
            jax.ShapeDtypeStruct((NR, F), jnp.float32),
        ],
        mesh=_sc_mesh(),
        compiler_params=pltpu.CompilerParams(needs_layout_passes=False),
        scratch_types=[
            pltpu.VMEM((N,), jnp.int32),
            pltpu.VMEM((MYCAP,), jnp.int32),
            pltpu.VMEM((MYCAP,), jnp.int32),
            pltpu.VMEM((NSUB * 16,), jnp.int32),
            pltpu.VMEM((C,), jnp.int32),
            pltpu.VMEM((C,), jnp.int32),
            pltpu.VMEM((C,), jnp.int32),
            pltpu.VMEM((16,), jnp.int32),
            pltpu.VMEM((TM,), jnp.int32),
            pltpu.VMEM((TM, F), jnp.float32),
            pltpu.VMEM((16,), jnp.int32),
            pltpu.VMEM((PG, F), jnp.float32),
            pltpu.VMEM_SHARED((NSUB * 16,), jnp.int32),
            pltpu.SemaphoreType.DMA,
        ],
    )
    perm, off, nblk, pcnt, xr = route(ids, x)

    yr = pl.pallas_call(
        _mm_body,
        grid_spec=pltpu.PrefetchScalarGridSpec(
            num_scalar_prefetch=2,
            grid=(C,),
            in_specs=[
                pl.BlockSpec((NR, F), lambda c, o, nb: (0, 0)),
                pl.BlockSpec((1, F, F), lambda c, o, nb: (c, 0, 0)),
                pl.BlockSpec((1, 1, F), lambda c, o, nb: (c, 0, 0)),
            ],
            out_specs=pl.BlockSpec((NR, F), lambda c, o, nb: (0, 0)),
        ),
        out_shape=jax.ShapeDtypeStruct((NR, F), jnp.float32),
    )(off, nblk, xr, W, b.reshape(C, 1, F))

    unroute = pl.kernel(
        _unroute_body,
        out_type=jax.ShapeDtypeStruct((N + PG, F), jnp.float32),
        mesh=_sc_mesh(),
        compiler_params=pltpu.CompilerParams(needs_layout_passes=False),
        scratch_types=[
            pltpu.VMEM((C,), jnp.int32),
            pltpu.VMEM((C,), jnp.int32),
            pltpu.VMEM((TM,), jnp.int32),
            pltpu.VMEM((TM, F), jnp.float32),
            pltpu.VMEM((16,), jnp.int32),
            pltpu.VMEM((PG, F), jnp.float32),
            pltpu.SemaphoreType.DMA,
        ],
    )
    ypad = unroute(perm, off, pcnt, yr)
    return ypad[:N]
